# baseline (speedup 1.0000x reference)
import functools

import jax
import jax.numpy as jnp
from jax import lax
from jax.experimental import pallas as pl
from jax.experimental.pallas import tpu as pltpu
from jax.experimental.pallas import tpu_sc as plsc

VOCAB = 1000000
EMBED_DIM = 64
BATCH = 16384

_INFO = plsc.get_sparse_core_info()
_NC = _INFO.num_cores
_NS = _INFO.num_subcores
_NW = _NC * _NS
_B_PER_W = BATCH // _NW
_GROUP = 16
_NGRP = _B_PER_W // _GROUP

_mesh = plsc.VectorSubcoreMesh(core_axis_name="c", subcore_axis_name="s")


@functools.partial(
    pl.kernel,
    mesh=_mesh,
    out_type=jax.ShapeDtypeStruct((BATCH, EMBED_DIM), jnp.float32),
    scratch_types=[
        pltpu.VMEM((_B_PER_W,), jnp.int32),
        pltpu.VMEM((_B_PER_W, EMBED_DIM), jnp.float32),
        pltpu.SemaphoreType.DMA,
    ],
    compiler_params=pltpu.CompilerParams(skip_device_barrier=True),
)
def _sc_gather(idx_hbm, table_hbm, out_hbm, idx_v, rows_v, sem):
    wid = lax.axis_index("s") * _NC + lax.axis_index("c")
    base = wid * _B_PER_W
    pltpu.sync_copy(idx_hbm.at[wid], idx_v)

    def body(g, carry):
        vec = idx_v[pl.ds(g * _GROUP, _GROUP)]
        copies = []
        for j in range(_GROUP):
            i = vec[j]
            copies.append(
                pltpu.async_copy(
                    table_hbm.at[pl.ds(i, 1)],
                    rows_v.at[pl.ds(g * _GROUP + j, 1)],
                    sem,
                )
            )
        for c in copies:
            c.wait()
        return carry

    lax.fori_loop(0, _NGRP, body, 0)
    pltpu.sync_copy(rows_v, out_hbm.at[pl.ds(base, _B_PER_W)])


def kernel(product_ids, table):
    idx = product_ids.astype(jnp.int32).reshape(_NW, _B_PER_W)
    return _sc_gather(idx, table)


# speedup vs baseline: 1.4507x; 1.4507x over previous
import functools

import jax
import jax.numpy as jnp
from jax import lax
from jax.experimental import pallas as pl
from jax.experimental.pallas import tpu as pltpu
from jax.experimental.pallas import tpu_sc as plsc

VOCAB = 1000000
EMBED_DIM = 64
BATCH = 16384

_INFO = plsc.get_sparse_core_info()
_NC = _INFO.num_cores
_NS = _INFO.num_subcores
_NW = _NC * _NS
_B_PER_W = BATCH // _NW
_GROUP = 16
_NGRP = _B_PER_W // _GROUP

_mesh = plsc.VectorSubcoreMesh(core_axis_name="c", subcore_axis_name="s")


@functools.partial(
    pl.kernel,
    mesh=_mesh,
    out_type=jax.ShapeDtypeStruct((BATCH // 8, 8, EMBED_DIM), jnp.float32),
    scratch_types=[
        pltpu.VMEM((_B_PER_W,), jnp.int32),
        pltpu.VMEM((_B_PER_W // 8, 8, EMBED_DIM), jnp.float32),
        pltpu.SemaphoreType.DMA,
    ],
)
def _sc_gather(idx_hbm, table_hbm, out_hbm, idx_v, rows_v, sem):
    wid = lax.axis_index("s") * _NC + lax.axis_index("c")
    base = wid * (_B_PER_W // 8)
    pltpu.sync_copy(idx_hbm.at[wid], idx_v)

    def body(g, carry):
        vec = idx_v[pl.ds(g * _GROUP, _GROUP)]
        copies = []
        for j in range(_GROUP):
            i = vec[j]
            copies.append(
                pltpu.async_copy(
                    table_hbm.at[i >> 3, pl.ds(i & 7, 1)],
                    rows_v.at[g * 2 + (j >> 3), pl.ds(j & 7, 1)],
                    sem,
                )
            )
        for c in copies:
            c.wait()
        return carry

    lax.fori_loop(0, _NGRP, body, 0)
    pltpu.sync_copy(rows_v, out_hbm.at[pl.ds(base, _B_PER_W // 8)])


def kernel(product_ids, table):
    idx = product_ids.astype(jnp.int32).reshape(_NW, _B_PER_W)
    table3 = table.reshape(VOCAB // 8, 8, EMBED_DIM)
    out = _sc_gather(idx, table3)
    return out.reshape(BATCH, EMBED_DIM)


# speedup vs baseline: 1.9065x; 1.3141x over previous
import functools

import jax
import jax.numpy as jnp
from jax import lax
from jax.experimental import pallas as pl
from jax.experimental.pallas import tpu as pltpu
from jax.experimental.pallas import tpu_sc as plsc

VOCAB = 1000000
EMBED_DIM = 64
BATCH = 16384

_INFO = plsc.get_sparse_core_info()
_NC = _INFO.num_cores
_NS = _INFO.num_subcores
_NW = _NC * _NS
_B_PER_W = BATCH // _NW
_L = 128
_VFULL = (VOCAB // _L) * _L
_LAST = _VFULL // _L - 1
_NTAIL = VOCAB - _VFULL
_GROUP = 4
_NBUF = 2 * _GROUP
_NGRP = _B_PER_W // _GROUP
_RINGW = _NBUF * _L
_TAILBASE = _RINGW

_mesh = plsc.VectorSubcoreMesh(core_axis_name="c", subcore_axis_name="s")


@functools.partial(
    pl.kernel,
    mesh=_mesh,
    out_type=jax.ShapeDtypeStruct((EMBED_DIM, BATCH), jnp.float32),
    scratch_types=[
        pltpu.VMEM((_B_PER_W + 16,), jnp.int32),
        pltpu.VMEM((EMBED_DIM, _RINGW + _NTAIL), jnp.float32),
        pltpu.VMEM((EMBED_DIM, _B_PER_W), jnp.float32),
        pltpu.SemaphoreType.DMA,
    ],
    compiler_params=pltpu.CompilerParams(needs_layout_passes=False),
)
def _sc_gather(idx_hbm, table_hbm, tail_hbm, out_hbm, idx_v, ring_v, cols_v,
               gsem):
    wid = lax.axis_index("s") * _NC + lax.axis_index("c")
    base = wid * _B_PER_W
    pltpu.sync_copy(idx_hbm.at[wid], idx_v.at[pl.ds(0, _B_PER_W)])
    pltpu.sync_copy(tail_hbm, ring_v.at[:, pl.ds(_TAILBASE, _NTAIL)])

    row_iota = lax.iota(jnp.int32, 16)

    def load_index(p):
        return idx_v[pl.ds(p, 16)][0]

    def fire(p, slot):
        i = load_index(p)
        j = jnp.minimum(i >> 7, _LAST)
        pltpu.async_copy(
            table_hbm.at[:, pl.ds(pl.multiple_of(j * _L, _L), _L)],
            ring_v.at[:, pl.ds(pl.multiple_of(slot * _L, _L), _L)],
            gsem,
        )

    def extract(p, slot):
        i = load_index(p)
        in_tail = i >= _VFULL
        off = jnp.where(in_tail, i - _VFULL + _TAILBASE,
                        slot * _L + (i & (_L - 1)))
        offsel = jnp.full((16,), off, jnp.int32)
        colsel = jnp.full((16,), p, jnp.int32)
        for k in range(EMBED_DIM // 16):
            rows = row_iota + (16 * k)
            vals = plsc.load_gather(ring_v, [rows, offsel])
            plsc.store_scatter(cols_v, [rows, colsel], vals)

    for p in range(_GROUP):
        fire(p, p)

    def body(g, carry):
        half = (g % 2) * _GROUP
        nhalf = ((g + 1) % 2) * _GROUP

        @pl.when(g + 1 < _NGRP)
        def _():
            for q in range(_GROUP):
                fire((g + 1) * _GROUP + q, nhalf + q)

        for q in range(_GROUP):
            pltpu.make_async_copy(
                table_hbm.at[:, pl.ds(0, _L)],
                ring_v.at[:, pl.ds(0, _L)],
                gsem,
            ).wait()
        for q in range(_GROUP):
            extract(g * _GROUP + q, half + q)
        return carry

    lax.fori_loop(0, _NGRP, body, 0)
    pltpu.sync_copy(cols_v, out_hbm.at[:, pl.ds(base, _B_PER_W)])


def kernel(product_ids, table):
    idx = product_ids.astype(jnp.int32).reshape(_NW, _B_PER_W)
    tail_t = table[_VFULL:, :].T
    out_t = _sc_gather(idx, table.T, tail_t)
    return out_t.T


# speedup vs baseline: 2.2396x; 1.1748x over previous
import functools

import jax
import jax.numpy as jnp
from jax import lax
from jax.experimental import pallas as pl
from jax.experimental.pallas import tpu as pltpu
from jax.experimental.pallas import tpu_sc as plsc

VOCAB = 1000000
EMBED_DIM = 64
BATCH = 16384

_INFO = plsc.get_sparse_core_info()
_NC = _INFO.num_cores
_NS = _INFO.num_subcores
_NW = _NC * _NS
_L = 128
_VFULL = (VOCAB // _L) * _L
_NTAIL = VOCAB - _VFULL
_CW = 256
_RPW = 31232
_NCH = _RPW // _CW
_NCH31 = (_VFULL - 31 * _RPW) // _CW
_NBKT = _NCH31 + 1
_D = 32
_NRING = 4
_TAILBASE = _NRING * _CW
_NSCAN = BATCH // 16

_mesh = plsc.VectorSubcoreMesh(core_axis_name="c", subcore_axis_name="s")


@functools.partial(
    pl.kernel,
    mesh=_mesh,
    out_type=jax.ShapeDtypeStruct((BATCH, EMBED_DIM), jnp.float32),
    scratch_types=[
        pltpu.VMEM((BATCH + 16,), jnp.int32),
        pltpu.VMEM((BATCH + 16,), jnp.int32),
        pltpu.VMEM((EMBED_DIM, _NRING * _CW + _NTAIL), jnp.float32),
        pltpu.VMEM((_NBKT * _D + 16,), jnp.int32),
        pltpu.VMEM((_NBKT + 16,), jnp.int32),
        pltpu.VMEM((_NBKT + 16,), jnp.int32),
        pltpu.VMEM((16, EMBED_DIM), jnp.float32),
        pltpu.SemaphoreType.DMA,
        pltpu.SemaphoreType.DMA,
    ],
    compiler_params=pltpu.CompilerParams(needs_layout_passes=False),
)
def _sc_gather(idx_hbm, table_hbm, tail_hbm, out_hbm, idx_v, own_v, ring_v,
               bkt_v, cnt_v, cnt2_v, row_v, csem, wsem):
    wid = lax.axis_index("s") * _NC + lax.axis_index("c")
    base = wid * _RPW
    is31 = wid == _NW - 1
    nch = jnp.where(is31, _NCH31, _NCH)
    bound = jnp.where(is31, VOCAB, base + _RPW)

    row_iota = lax.iota(jnp.int32, 16)
    lane0 = row_iota == 0

    def sload(ref, p):
        return ref[pl.ds(p, 16)][0]

    def fire_chunk(c):
        pltpu.async_copy(
            table_hbm.at[:, pl.ds(pl.multiple_of(base + c * _CW, _L), _CW)],
            ring_v.at[:, pl.ds(pl.multiple_of((c % _NRING) * _CW, _L), _CW)],
            csem,
        )

    def drain_chunk():
        pltpu.make_async_copy(
            table_hbm.at[:, pl.ds(0, _CW)],
            ring_v.at[:, pl.ds(0, _CW)],
            csem,
        ).wait()

    def drain_row():
        pltpu.make_async_copy(
            out_hbm.at[pl.ds(0, 1)],
            row_v.at[pl.ds(0, 1)],
            wsem,
        ).wait()

    def extract(b, i, off, wq):
        s = wq & 15

        @pl.when(wq >= 16)
        def _():
            drain_row()

        for k in range(EMBED_DIM // 16):
            rows = row_iota + (16 * k)
            vals = plsc.load_gather(ring_v, [rows, jnp.full((16,), off, jnp.int32)])
            plsc.store_scatter(row_v, [jnp.full((16,), s, jnp.int32), rows], vals)
        pltpu.async_copy(row_v.at[pl.ds(s, 1)], out_hbm.at[pl.ds(b, 1)], wsem)
        return wq + 1

    pltpu.sync_copy(idx_hbm, idx_v.at[pl.ds(0, BATCH)])
    pltpu.sync_copy(tail_hbm, ring_v.at[:, pl.ds(_TAILBASE, _NTAIL)])
    fire_chunk(0)
    fire_chunk(1)
    fire_chunk(2)

    zeros = jnp.zeros((16,), jnp.int32)
    for k in range((_NBKT + 15) // 16):
        cnt_v[pl.ds(16 * k, 16)] = zeros
        cnt2_v[pl.ds(16 * k, 16)] = zeros

    def scan_body(v, on):
        ivec = idx_v[pl.ds(v * 16, 16)]
        mine = (ivec >= base) & (ivec < bound)
        plsc.store_compressed(own_v.at[pl.ds(on, 16)], row_iota + v * 16, mask=mine)
        return on + plsc.all_reduce_population_count(mine)[0]

    on = lax.fori_loop(0, _NSCAN, scan_body, jnp.int32(0), unroll=8)

    def bkt_body(p, carry):
        b = sload(own_v, p)
        i = sload(idx_v, b)
        c = (i - base) >> 8
        cnt = sload(cnt_v, c)
        plsc.store_scatter(cnt_v, [jnp.full((16,), c, jnp.int32)],
                           jnp.full((16,), cnt + 1, jnp.int32), mask=lane0)
        slot = c * _D + jnp.minimum(cnt, _D - 1)
        plsc.store_scatter(bkt_v, [jnp.full((16,), slot, jnp.int32)],
                           jnp.full((16,), b, jnp.int32),
                           mask=lane0 & (cnt < _D))
        return carry

    lax.fori_loop(0, on, bkt_body, 0)

    def chunk_body(c, wq):
        @pl.when(c + 3 < nch)
        def _():
            fire_chunk(c + 3)

        drain_chunk()
        slotbase = (c % _NRING) * _CW

        def ex_body(q, wq):
            b = sload(bkt_v, c * _D + q)
            i = sload(idx_v, b)
            return extract(b, i, slotbase + (i - base - c * _CW), wq)

        return lax.fori_loop(0, jnp.minimum(sload(cnt_v, c), _D), ex_body, wq)

    wq = lax.fori_loop(0, nch, chunk_body, jnp.int32(0))

    def tail_body(q, wq):
        b = sload(bkt_v, _NCH31 * _D + q)
        i = sload(idx_v, b)
        return extract(b, i, _TAILBASE + (i - _VFULL), wq)

    wq = lax.fori_loop(0, jnp.minimum(sload(cnt_v, _NCH31), _D), tail_body, wq)

    lax.fori_loop(0, jnp.minimum(wq, 16), lambda d, z: (drain_row(), z)[1], 0)

    def ovf_body(p, carry):
        b = sload(own_v, p)
        i = sload(idx_v, b)
        c = (i - base) >> 8
        cnt = sload(cnt2_v, c)
        plsc.store_scatter(cnt2_v, [jnp.full((16,), c, jnp.int32)],
                           jnp.full((16,), cnt + 1, jnp.int32), mask=lane0)

        @pl.when(cnt >= _D)
        def _():
            j = jnp.minimum(i >> 7, _VFULL // _L - 1)
            pltpu.sync_copy(
                table_hbm.at[:, pl.ds(pl.multiple_of(j * _L, _L), _L)],
                ring_v.at[:, pl.ds(0, _L)],
            )
            in_tail = i >= _VFULL
            off = jnp.where(in_tail, _TAILBASE + (i - _VFULL), i & (_L - 1))
            for k in range(EMBED_DIM // 16):
                rows = row_iota + (16 * k)
                vals = plsc.load_gather(ring_v, [rows, jnp.full((16,), off, jnp.int32)])
                plsc.store_scatter(row_v, [jnp.full((16,), 0, jnp.int32), rows], vals)
            pltpu.sync_copy(row_v.at[pl.ds(0, 1)], out_hbm.at[pl.ds(b, 1)])

        return carry

    lax.fori_loop(0, on, ovf_body, 0)


def kernel(product_ids, table):
    idx = product_ids.astype(jnp.int32)
    tail_t = table[_VFULL:, :].T
    return _sc_gather(idx, table.T, tail_t)


# speedup vs baseline: 2.5782x; 1.1512x over previous
import functools

import jax
import jax.numpy as jnp
from jax import lax
from jax.experimental import pallas as pl
from jax.experimental.pallas import tpu as pltpu
from jax.experimental.pallas import tpu_sc as plsc

VOCAB = 1000000
EMBED_DIM = 64
BATCH = 16384

_INFO = plsc.get_sparse_core_info()
_NC = _INFO.num_cores
_NS = _INFO.num_subcores
_NW = _NC * _NS
_L = 128
_VFULL = (VOCAB // _L) * _L
_NTAIL = VOCAB - _VFULL
_CW = 256
_RPW = 31232
_NCH = _RPW // _CW
_NCH31 = (_VFULL - 31 * _RPW) // _CW
_NBKT = _NCH31 + 1
_D = 32
_NRING = 5
_TAILBASE = _NRING * _CW
_NSCAN = BATCH // 16

_mesh = plsc.VectorSubcoreMesh(core_axis_name="c", subcore_axis_name="s")


@functools.partial(
    pl.kernel,
    mesh=_mesh,
    out_type=jax.ShapeDtypeStruct((BATCH, EMBED_DIM), jnp.float32),
    scratch_types=[
        pltpu.VMEM((BATCH + 16,), jnp.int32),
        pltpu.VMEM((BATCH + 16,), jnp.int32),
        pltpu.VMEM((EMBED_DIM, _NRING * _CW + _NTAIL), jnp.float32),
        pltpu.VMEM((_NBKT * _D + 16,), jnp.int32),
        pltpu.VMEM((_NBKT + 16,), jnp.int32),
        pltpu.VMEM((_NBKT + 16,), jnp.int32),
        pltpu.VMEM((16, EMBED_DIM), jnp.float32),
        pltpu.SemaphoreType.DMA,
        pltpu.SemaphoreType.DMA,
    ],
    compiler_params=pltpu.CompilerParams(needs_layout_passes=False),
)
def _sc_gather(idx_hbm, table_hbm, tail_hbm, out_hbm, idx_v, own_v, ring_v,
               bkt_v, cnt_v, cnt2_v, row_v, csem, wsem):
    wid = lax.axis_index("s") * _NC + lax.axis_index("c")
    base = wid * _RPW
    is31 = wid == _NW - 1
    nch = jnp.where(is31, _NCH31, _NCH)
    bound = jnp.where(is31, VOCAB, base + _RPW)

    row_iota = lax.iota(jnp.int32, 16)
    lane0 = row_iota == 0

    def sload(ref, p):
        return ref[pl.ds(p, 16)][0]

    def fire_chunk(c):
        pltpu.async_copy(
            table_hbm.at[:, pl.ds(pl.multiple_of(base + c * _CW, _L), _CW)],
            ring_v.at[:, pl.ds(pl.multiple_of((c % _NRING) * _CW, _L), _CW)],
            csem,
        )

    def drain_chunk():
        pltpu.make_async_copy(
            table_hbm.at[:, pl.ds(0, _CW)],
            ring_v.at[:, pl.ds(0, _CW)],
            csem,
        ).wait()

    def drain_row():
        pltpu.make_async_copy(
            out_hbm.at[pl.ds(0, 1)],
            row_v.at[pl.ds(0, 1)],
            wsem,
        ).wait()

    def extract(b, i, off, wq):
        s = wq & 15

        @pl.when(wq >= 16)
        def _():
            drain_row()

        for k in range(EMBED_DIM // 16):
            rows = row_iota + (16 * k)
            vals = plsc.load_gather(ring_v, [rows, jnp.full((16,), off, jnp.int32)])
            plsc.store_scatter(row_v, [jnp.full((16,), s, jnp.int32), rows], vals)
        pltpu.async_copy(row_v.at[pl.ds(s, 1)], out_hbm.at[pl.ds(b, 1)], wsem)
        return wq + 1

    pltpu.sync_copy(idx_hbm, idx_v.at[pl.ds(0, BATCH)])
    pltpu.sync_copy(tail_hbm, ring_v.at[:, pl.ds(_TAILBASE, _NTAIL)])
    for c0 in range(_NRING - 1):
        fire_chunk(c0)

    zeros = jnp.zeros((16,), jnp.int32)
    for k in range((_NBKT + 15) // 16):
        cnt_v[pl.ds(16 * k, 16)] = zeros
        cnt2_v[pl.ds(16 * k, 16)] = zeros

    def scan_body(v, on):
        ivec = idx_v[pl.ds(v * 16, 16)]
        mine = (ivec >= base) & (ivec < bound)
        plsc.store_compressed(own_v.at[pl.ds(on, 16)], row_iota + v * 16, mask=mine)
        return on + plsc.all_reduce_population_count(mine)[0]

    on = lax.fori_loop(0, _NSCAN, scan_body, jnp.int32(0), unroll=16)

    def bkt_body(p, carry):
        b = sload(own_v, p)
        i = sload(idx_v, b)
        c = (i - base) >> 8
        cnt = sload(cnt_v, c)
        plsc.store_scatter(cnt_v, [jnp.full((16,), c, jnp.int32)],
                           jnp.full((16,), cnt + 1, jnp.int32), mask=lane0)
        slot = c * _D + jnp.minimum(cnt, _D - 1)
        plsc.store_scatter(bkt_v, [jnp.full((16,), slot, jnp.int32)],
                           jnp.full((16,), b, jnp.int32),
                           mask=lane0 & (cnt < _D))
        return carry | (cnt >= _D - 1).astype(jnp.int32)

    ovf_any = lax.fori_loop(0, on, bkt_body, jnp.int32(0))

    def chunk_body(c, wq):
        @pl.when(c + _NRING - 1 < nch)
        def _():
            fire_chunk(c + _NRING - 1)

        drain_chunk()
        slotbase = (c % _NRING) * _CW

        def ex_body(q, wq):
            b = sload(bkt_v, c * _D + q)
            i = sload(idx_v, b)
            return extract(b, i, slotbase + (i - base - c * _CW), wq)

        return lax.fori_loop(0, jnp.minimum(sload(cnt_v, c), _D), ex_body, wq)

    wq = lax.fori_loop(0, nch, chunk_body, jnp.int32(0))

    def tail_body(q, wq):
        b = sload(bkt_v, _NCH31 * _D + q)
        i = sload(idx_v, b)
        return extract(b, i, _TAILBASE + (i - _VFULL), wq)

    wq = lax.fori_loop(0, jnp.minimum(sload(cnt_v, _NCH31), _D), tail_body, wq)

    lax.fori_loop(0, jnp.minimum(wq, 16), lambda d, z: (drain_row(), z)[1], 0)

    def ovf_body(p, carry):
        b = sload(own_v, p)
        i = sload(idx_v, b)
        c = (i - base) >> 8
        cnt = sload(cnt2_v, c)
        plsc.store_scatter(cnt2_v, [jnp.full((16,), c, jnp.int32)],
                           jnp.full((16,), cnt + 1, jnp.int32), mask=lane0)

        @pl.when(cnt >= _D)
        def _():
            j = jnp.minimum(i >> 7, _VFULL // _L - 1)
            pltpu.sync_copy(
                table_hbm.at[:, pl.ds(pl.multiple_of(j * _L, _L), _L)],
                ring_v.at[:, pl.ds(0, _L)],
            )
            in_tail = i >= _VFULL
            off = jnp.where(in_tail, _TAILBASE + (i - _VFULL), i & (_L - 1))
            for k in range(EMBED_DIM // 16):
                rows = row_iota + (16 * k)
                vals = plsc.load_gather(ring_v, [rows, jnp.full((16,), off, jnp.int32)])
                plsc.store_scatter(row_v, [jnp.full((16,), 0, jnp.int32), rows], vals)
            pltpu.sync_copy(row_v.at[pl.ds(0, 1)], out_hbm.at[pl.ds(b, 1)])

        return carry

    @pl.when(ovf_any > 0)
    def _():
        lax.fori_loop(0, on, ovf_body, 0)


def kernel(product_ids, table):
    idx = product_ids.astype(jnp.int32)
    tail_t = table[_VFULL:, :].T
    return _sc_gather(idx, table.T, tail_t)
